# baseline (device time: 105164 ns/iter reference)
import functools

import jax
import jax.numpy as jnp
from jax import lax
from jax.experimental import pallas as pl
from jax.experimental.pallas import tpu as pltpu

N_DEV = 8


def kernel(x, Wq, K_ext, V_ext, Wo):
    B, Sq, Dm = x.shape
    _, Skv, Hq, Dh = K_ext.shape
    Dq = Wq.shape[1]
    Do = Wo.shape[1]

    def body(x_ref, wq_ref, k_ref, v_ref, wo_ref, out_ref,
             kv_ref, send_sem, recv_sem):
        my = lax.axis_index("i")
        left = lax.rem(my - 1 + N_DEV, N_DEV)
        right = lax.rem(my + 1, N_DEV)

        barrier_sem = pltpu.get_barrier_semaphore()
        for nbr in (left, right):
            pl.semaphore_signal(
                barrier_sem, inc=1,
                device_id=(nbr,), device_id_type=pl.DeviceIdType.MESH,
            )
        pl.semaphore_wait(barrier_sem, 2)

        @pl.when(my == 0)
        def _():
            kv_ref[0] = k_ref[...].astype(jnp.bfloat16)
            kv_ref[1] = v_ref[...].astype(jnp.bfloat16)

        recv = pltpu.make_async_remote_copy(
            src_ref=kv_ref, dst_ref=kv_ref,
            send_sem=send_sem, recv_sem=recv_sem,
            device_id=(left,), device_id_type=pl.DeviceIdType.MESH,
        )

        @pl.when(my > 0)
        def _():
            recv.wait_recv()

        @pl.when(my < N_DEV - 1)
        def _():
            send = pltpu.make_async_remote_copy(
                src_ref=kv_ref, dst_ref=kv_ref,
                send_sem=send_sem, recv_sem=recv_sem,
                device_id=(right,), device_id_type=pl.DeviceIdType.MESH,
            )
            send.start()
            send.wait_send()

        qb = lax.broadcasted_iota(jnp.int32, (Sq, Skv), 0) // 64
        kb = lax.broadcasted_iota(jnp.int32, (Sq, Skv), 1) // 64
        mask = kb <= qb

        wq = wq_ref[...].astype(jnp.bfloat16)
        wo = wo_ref[...].astype(jnp.bfloat16)

        for b in range(B):
            xb = x_ref[b].astype(jnp.bfloat16)
            q = jnp.dot(xb, wq, preferred_element_type=jnp.float32)
            qbf = q.astype(jnp.bfloat16)
            acc = jnp.zeros((Sq, Do), jnp.float32)
            for h in range(Hq):
                qh = qbf[:, h * Dh:(h + 1) * Dh]
                kh = kv_ref[0, b, :, h, :]
                vh = kv_ref[1, b, :, h, :]
                s = lax.dot_general(
                    qh, kh, (((1,), (1,)), ((), ())),
                    preferred_element_type=jnp.float32,
                ) * 0.125
                s = jnp.where(mask, s, -1e9)
                m = jnp.max(s, axis=1, keepdims=True)
                w = jnp.exp(s - m)
                w = w / jnp.sum(w, axis=1, keepdims=True)
                ctx = jnp.dot(w.astype(jnp.bfloat16), vh,
                              preferred_element_type=jnp.float32)
                acc = acc + jnp.dot(
                    ctx.astype(jnp.bfloat16), wo[h * Dh:(h + 1) * Dh, :],
                    preferred_element_type=jnp.float32,
                )
            out_ref[b] = acc

        @functools.partial(pl.run_scoped, sem2=pltpu.SemaphoreType.REGULAR)
        def _(sem2):
            for nbr in (left, right):
                pl.semaphore_signal(
                    sem2, inc=1,
                    device_id=(nbr,), device_id_type=pl.DeviceIdType.MESH,
                )
            pl.semaphore_wait(sem2, 2)

    return pl.pallas_call(
        body,
        out_shape=jax.ShapeDtypeStruct((B, Sq, Do), jnp.float32),
        in_specs=[pl.BlockSpec(memory_space=pltpu.VMEM)] * 5,
        out_specs=pl.BlockSpec(memory_space=pltpu.VMEM),
        scratch_shapes=[
            pltpu.VMEM((2, B, Skv, Hq, Dh), jnp.bfloat16),
            pltpu.SemaphoreType.DMA,
            pltpu.SemaphoreType.DMA,
        ],
        compiler_params=pltpu.CompilerParams(collective_id=0),
    )(x, Wq, K_ext, V_ext, Wo)


# device time: 14488 ns/iter; 7.2587x vs baseline; 7.2587x over previous
import functools

import jax
import jax.numpy as jnp
from jax import lax
from jax.experimental import pallas as pl
from jax.experimental.pallas import tpu as pltpu

N_DEV = 8


def kernel(x, Wq, K_ext, V_ext, Wo):
    B, Sq, Dm = x.shape
    _, Skv, Hq, Dh = K_ext.shape
    Dq = Wq.shape[1]
    Do = Wo.shape[1]

    def body(x_ref, wq_ref, k_ref, v_ref, wo_ref, out_ref,
             kv_ref, send_sem, recv_sem):
        my = lax.axis_index("i")
        left = lax.rem(my - 1 + N_DEV, N_DEV)
        right = lax.rem(my + 1, N_DEV)

        barrier_sem = pltpu.get_barrier_semaphore()
        for nbr in (left, right):
            pl.semaphore_signal(
                barrier_sem, inc=1,
                device_id=(nbr,), device_id_type=pl.DeviceIdType.MESH,
            )
        pl.semaphore_wait(barrier_sem, 2)

        kv_ref[0] = k_ref[...].astype(jnp.bfloat16)
        kv_ref[1] = v_ref[...].astype(jnp.bfloat16)

        qb = lax.broadcasted_iota(jnp.int32, (Sq, Skv), 0) // 64
        kb = lax.broadcasted_iota(jnp.int32, (Sq, Skv), 1) // 64
        mask = kb <= qb

        wq = wq_ref[...].astype(jnp.bfloat16)
        wo = wo_ref[...].astype(jnp.bfloat16)

        for b in range(B):
            xb = x_ref[b].astype(jnp.bfloat16)
            q = jnp.dot(xb, wq, preferred_element_type=jnp.float32)
            qbf = q.astype(jnp.bfloat16)
            acc = jnp.zeros((Sq, Do), jnp.float32)
            for h in range(Hq):
                qh = qbf[:, h * Dh:(h + 1) * Dh]
                kh = kv_ref[0, b, :, h, :]
                vh = kv_ref[1, b, :, h, :]
                s = lax.dot_general(
                    qh, kh, (((1,), (1,)), ((), ())),
                    preferred_element_type=jnp.float32,
                ) * 0.125
                s = jnp.where(mask, s, -1e9)
                m = jnp.max(s, axis=1, keepdims=True)
                w = jnp.exp(s - m)
                w = w / jnp.sum(w, axis=1, keepdims=True)
                ctx = jnp.dot(w.astype(jnp.bfloat16), vh,
                              preferred_element_type=jnp.float32)
                acc = acc + jnp.dot(
                    ctx.astype(jnp.bfloat16), wo[h * Dh:(h + 1) * Dh, :],
                    preferred_element_type=jnp.float32,
                )
            out_ref[b] = acc

        @functools.partial(pl.run_scoped, sem2=pltpu.SemaphoreType.REGULAR)
        def _(sem2):
            for nbr in (left, right):
                pl.semaphore_signal(
                    sem2, inc=1,
                    device_id=(nbr,), device_id_type=pl.DeviceIdType.MESH,
                )
            pl.semaphore_wait(sem2, 2)

    return pl.pallas_call(
        body,
        out_shape=jax.ShapeDtypeStruct((B, Sq, Do), jnp.float32),
        in_specs=[pl.BlockSpec(memory_space=pltpu.VMEM)] * 5,
        out_specs=pl.BlockSpec(memory_space=pltpu.VMEM),
        scratch_shapes=[
            pltpu.VMEM((2, B, Skv, Hq, Dh), jnp.bfloat16),
            pltpu.SemaphoreType.DMA,
            pltpu.SemaphoreType.DMA,
        ],
        compiler_params=pltpu.CompilerParams(collective_id=0),
    )(x, Wq, K_ext, V_ext, Wo)
